# baseline (speedup 1.0000x reference)
import jax
import jax.numpy as jnp
from jax.experimental import pallas as pl
from jax.scipy.linalg import block_diag

BN = 256
GRP = 2
R0 = 40
R2 = 16


def _fluid_solid_kernel(mask_ref, cdT_ref, cT_ref,
                        Wa_ref, Wc_ref, b0_ref,
                        W1b_ref, W2b_ref, W3g_ref,
                        fb3_ref, sb3_ref, out_ref):
    cdT = cdT_ref[...]
    m = cdT.shape[1]
    ptype = cdT[6:7, :]
    tf_row = 1.0 - ptype
    ts_row = ptype

    afs = jnp.dot(Wa_ref[...], cdT[0:6, :])
    bfs = jnp.dot(Wc_ref[...], cT_ref[0]) + b0_ref[...]

    mask_blk = mask_ref[...].astype(jnp.float32)

    wf = mask_blk * tf_row
    ws = mask_blk * ts_row

    zsel = jnp.zeros((GRP, R2 - 12, m), jnp.float32)
    ones_row = jnp.ones((1, m), jnp.float32)
    outs = []
    for g in range(BN // GRP):
        x0 = jnp.concatenate(
            [afs + bfs[:, g * GRP + i:g * GRP + i + 1] for i in range(GRP)]
            + [ones_row], axis=0)
        x0 = jnp.tanh(x0).astype(jnp.bfloat16)
        x1 = jnp.tanh(jnp.dot(W1b_ref[...], x0,
                              preferred_element_type=jnp.float32))
        x1 = jnp.concatenate([x1, ones_row], axis=0).astype(jnp.bfloat16)
        x2 = jnp.tanh(jnp.dot(W2b_ref[...], x1,
                              preferred_element_type=jnp.float32))

        wsel = jnp.concatenate(
            [jnp.broadcast_to(wf[g * GRP:(g + 1) * GRP, None, :], (GRP, 6, m)),
             jnp.broadcast_to(ws[g * GRP:(g + 1) * GRP, None, :], (GRP, 6, m)),
             zsel], axis=1).reshape(R2 * GRP, m)
        s = jnp.sum(x2 * wsel, axis=1, keepdims=True)
        outs.append(jnp.dot(W3g_ref[...], s).reshape(GRP, 3))
    out = jnp.concatenate(outs, axis=0)

    cf = jnp.sum(wf, axis=1, keepdims=True)
    cs = jnp.sum(ws, axis=1, keepdims=True)
    out_ref[0] = out + cf * fb3_ref[...] + cs * sb3_ref[...]


def kernel(mask, center_particle, current_data,
           fW0, fb0, fW1, fb1, fW2, fb2, fW3, fb3,
           sW0, sb0, sW1, sb1, sW2, sb2, sW3, sb3):
    n, m = mask.shape
    maskf = mask.astype(jnp.int8)
    cdT = current_data.T
    grid = n // BN
    cTb = center_particle.T.reshape(6, grid, BN).transpose(1, 0, 2)

    z = lambda r, c: jnp.zeros((r, c), jnp.float32)
    Wa = jnp.concatenate(
        [fW0[0:6].T,
         jnp.concatenate([sW0[0:3].T, z(18, 3)], axis=1),
         z(R0 - 36, 6)], axis=0)
    Wc = jnp.concatenate(
        [jnp.concatenate([-fW0[0:3].T, fW0[6:9].T], axis=1),
         jnp.concatenate([-sW0[0:3].T, sW0[3:6].T], axis=1),
         z(R0 - 36, 6)], axis=0)
    b0 = jnp.concatenate([fb0, sb0, jnp.zeros(R0 - 36)]).reshape(R0, 1)

    W1fs = jnp.concatenate([block_diag(fW1.T, sW1.T), z(18, R0 - 36)],
                           axis=1)
    W1g = block_diag(*([W1fs] * GRP))
    b1 = jnp.tile(jnp.concatenate([fb1, sb1]), GRP).reshape(18 * GRP, 1)
    W1b = jnp.concatenate([W1g, b1 / jnp.tanh(1.0)], axis=1)
    W2fs = jnp.concatenate([block_diag(fW2.T, sW2.T), z(R2 - 12, 18)],
                           axis=0)
    W2g = block_diag(*([W2fs] * GRP))
    b2 = jnp.tile(jnp.concatenate([fb2, sb2, jnp.zeros(R2 - 12)]),
                  GRP).reshape(R2 * GRP, 1)
    W2b = jnp.concatenate([W2g, b2], axis=1)
    W3fs = jnp.concatenate([fW3.T, sW3.T, z(3, R2 - 12)], axis=1)
    W3g = block_diag(*([W3fs] * GRP))

    full = lambda shape: pl.BlockSpec(shape, lambda i: tuple(0 for _ in shape))
    out = pl.pallas_call(
        _fluid_solid_kernel,
        grid=(grid,),
        in_specs=[
            pl.BlockSpec((BN, m), lambda i: (i, 0)),
            full((7, m)),
            pl.BlockSpec((1, 6, BN), lambda i: (i, 0, 0)),
            full((R0, 6)), full((R0, 6)), full((R0, 1)),
            full((18 * GRP, R0 * GRP + 1)),
            full((R2 * GRP, 18 * GRP + 1)),
            full((3 * GRP, R2 * GRP)),
            full((1, 3)), full((1, 3)),
        ],
        out_specs=pl.BlockSpec((1, BN, 3), lambda i: (i, 0, 0)),
        out_shape=jax.ShapeDtypeStruct((grid, BN, 3), jnp.float32),
    )(maskf, cdT, cTb, Wa, Wc, b0,
      W1b.astype(jnp.bfloat16), W2b.astype(jnp.bfloat16), W3g,
      fb3.reshape(1, 3), sb3.reshape(1, 3))
    return out.reshape(n, 3)


# speedup vs baseline: 1.7373x; 1.7373x over previous
import jax
import jax.numpy as jnp
from jax.experimental import pallas as pl
from jax.scipy.linalg import block_diag

BN = 256
GRP = 8
R0 = 40
R2 = 16


def _fluid_solid_kernel(mask_ref, cdT_ref, cT_ref,
                        Wa_ref, Wc_ref, b0_ref,
                        W1b_ref, W2b_ref, W3g_ref,
                        fb3_ref, sb3_ref, out_ref):
    cdT = cdT_ref[...]
    m = cdT.shape[1]
    ptype = cdT[6:7, :]
    tf_row = 1.0 - ptype
    ts_row = ptype

    afs = jnp.dot(Wa_ref[...], cdT[0:6, :])
    bfs = jnp.dot(Wc_ref[...], cT_ref[0]) + b0_ref[...]

    mask_blk = mask_ref[...].astype(jnp.float32)

    wf = mask_blk * tf_row
    ws = mask_blk * ts_row

    zsel = jnp.zeros((GRP, R2 - 12, m), jnp.float32)
    ones_row = jnp.ones((1, m), jnp.float32)
    outs = []
    for g in range(BN // GRP):
        x0 = jnp.concatenate(
            [afs + bfs[:, g * GRP + i:g * GRP + i + 1] for i in range(GRP)]
            + [ones_row], axis=0)
        x0 = jnp.tanh(x0).astype(jnp.bfloat16)
        x1 = jnp.tanh(jnp.dot(W1b_ref[...], x0,
                              preferred_element_type=jnp.float32))
        x1 = jnp.concatenate([x1, ones_row], axis=0).astype(jnp.bfloat16)
        x2 = jnp.tanh(jnp.dot(W2b_ref[...], x1,
                              preferred_element_type=jnp.float32))

        wsel = jnp.concatenate(
            [jnp.broadcast_to(wf[g * GRP:(g + 1) * GRP, None, :], (GRP, 6, m)),
             jnp.broadcast_to(ws[g * GRP:(g + 1) * GRP, None, :], (GRP, 6, m)),
             zsel], axis=1).reshape(R2 * GRP, m)
        s = jnp.sum(x2 * wsel, axis=1, keepdims=True)
        outs.append(jnp.dot(W3g_ref[...], s).reshape(GRP, 3))
    out = jnp.concatenate(outs, axis=0)

    cf = jnp.sum(wf, axis=1, keepdims=True)
    cs = jnp.sum(ws, axis=1, keepdims=True)
    out_ref[0] = out + cf * fb3_ref[...] + cs * sb3_ref[...]


def kernel(mask, center_particle, current_data,
           fW0, fb0, fW1, fb1, fW2, fb2, fW3, fb3,
           sW0, sb0, sW1, sb1, sW2, sb2, sW3, sb3):
    n, m = mask.shape
    maskf = mask.astype(jnp.int8)
    cdT = current_data.T
    grid = n // BN
    cTb = center_particle.T.reshape(6, grid, BN).transpose(1, 0, 2)

    z = lambda r, c: jnp.zeros((r, c), jnp.float32)
    Wa = jnp.concatenate(
        [fW0[0:6].T,
         jnp.concatenate([sW0[0:3].T, z(18, 3)], axis=1),
         z(R0 - 36, 6)], axis=0)
    Wc = jnp.concatenate(
        [jnp.concatenate([-fW0[0:3].T, fW0[6:9].T], axis=1),
         jnp.concatenate([-sW0[0:3].T, sW0[3:6].T], axis=1),
         z(R0 - 36, 6)], axis=0)
    b0 = jnp.concatenate([fb0, sb0, jnp.zeros(R0 - 36)]).reshape(R0, 1)

    W1fs = jnp.concatenate([block_diag(fW1.T, sW1.T), z(18, R0 - 36)],
                           axis=1)
    W1g = block_diag(*([W1fs] * GRP))
    b1 = jnp.tile(jnp.concatenate([fb1, sb1]), GRP).reshape(18 * GRP, 1)
    W1b = jnp.concatenate([W1g, b1 / jnp.tanh(1.0)], axis=1)
    W2fs = jnp.concatenate([block_diag(fW2.T, sW2.T), z(R2 - 12, 18)],
                           axis=0)
    W2g = block_diag(*([W2fs] * GRP))
    b2 = jnp.tile(jnp.concatenate([fb2, sb2, jnp.zeros(R2 - 12)]),
                  GRP).reshape(R2 * GRP, 1)
    W2b = jnp.concatenate([W2g, b2], axis=1)
    W3fs = jnp.concatenate([fW3.T, sW3.T, z(3, R2 - 12)], axis=1)
    W3g = block_diag(*([W3fs] * GRP))

    full = lambda shape: pl.BlockSpec(shape, lambda i: tuple(0 for _ in shape))
    out = pl.pallas_call(
        _fluid_solid_kernel,
        grid=(grid,),
        in_specs=[
            pl.BlockSpec((BN, m), lambda i: (i, 0)),
            full((7, m)),
            pl.BlockSpec((1, 6, BN), lambda i: (i, 0, 0)),
            full((R0, 6)), full((R0, 6)), full((R0, 1)),
            full((18 * GRP, R0 * GRP + 1)),
            full((R2 * GRP, 18 * GRP + 1)),
            full((3 * GRP, R2 * GRP)),
            full((1, 3)), full((1, 3)),
        ],
        out_specs=pl.BlockSpec((1, BN, 3), lambda i: (i, 0, 0)),
        out_shape=jax.ShapeDtypeStruct((grid, BN, 3), jnp.float32),
    )(maskf, cdT, cTb, Wa, Wc, b0,
      W1b.astype(jnp.bfloat16), W2b.astype(jnp.bfloat16), W3g,
      fb3.reshape(1, 3), sb3.reshape(1, 3))
    return out.reshape(n, 3)
